# Initial kernel scaffold; baseline (speedup 1.0000x reference)
import jax
import jax.numpy as jnp
from jax.experimental import pallas as pl


def kernel(x, edge_index, batch, attr, W_l, b_l, W_r, W_lin, b_lin, size_src, size_dst):
    raise NotImplementedError("write your pallas kernel here")


# speedup vs baseline: 6.7789x; 6.7789x over previous
import functools

import jax
import jax.numpy as jnp
from jax import lax
from jax.experimental import pallas as pl
from jax.experimental.pallas import tpu as pltpu
from jax.experimental.pallas import tpu_sc as plsc

NC = 2
NS = 16
NW = NC * NS
CHUNK = 128


def _sc_agg(x, src, dst, batch, attr, zs, zc, ones):
    n_src, d = x.shape
    e = src.shape[0]
    n_dst = zs.shape[0]
    n_chunks = e // CHUNK
    stripe = 256
    s_tail = n_dst - (NS - 1) * stripe
    a_tail = n_dst - (NW - 1) * 128

    mesh = plsc.VectorSubcoreMesh(core_axis_name="c", subcore_axis_name="s")

    @functools.partial(
        pl.kernel,
        out_type=(
            jax.ShapeDtypeStruct((NC, n_dst, d), jnp.float32),
            jax.ShapeDtypeStruct((NC, n_dst, d), jnp.float32),
            jax.ShapeDtypeStruct((n_dst, d), jnp.float32),
        ),
        mesh=mesh,
        scratch_types=(
            pltpu.VMEM((CHUNK,), jnp.int32),
            pltpu.VMEM((CHUNK,), jnp.int32),
            pltpu.VMEM((CHUNK, d), jnp.float32),
            pltpu.VMEM((CHUNK, d), jnp.float32),
            pltpu.SemaphoreType.DMA,
            pltpu.VMEM_SHARED((n_dst, d), jnp.float32),
            pltpu.VMEM_SHARED((n_dst, d), jnp.float32),
        ),
    )
    def body(x_h, src_h, dst_h, batch_h, attr_h, zs_h, zc_h, ones_h,
             summed_o, cnt_o, attr_o,
             src_v, dst_v, rows_v, ones_v, sem, summed_sh, cnt_sh):
        c = lax.axis_index("c")
        s = lax.axis_index("s")
        wid = s * NC + c

        r0 = pl.multiple_of(s * stripe, stripe)

        @pl.when(s < NS - 1)
        def _zero_full():
            pltpu.sync_copy(zs_h.at[pl.ds(r0, stripe)],
                            summed_sh.at[pl.ds(r0, stripe)])
            pltpu.sync_copy(zc_h.at[pl.ds(r0, stripe)],
                            cnt_sh.at[pl.ds(r0, stripe)])

        @pl.when(s == NS - 1)
        def _zero_tail():
            t0 = (NS - 1) * stripe
            pltpu.sync_copy(zs_h.at[pl.ds(t0, s_tail)],
                            summed_sh.at[pl.ds(t0, s_tail)])
            pltpu.sync_copy(zc_h.at[pl.ds(t0, s_tail)],
                            cnt_sh.at[pl.ds(t0, s_tail)])

        pltpu.sync_copy(ones_h, ones_v)
        plsc.subcore_barrier()

        c0 = (n_chunks * wid) // NW
        c1 = (n_chunks * (wid + 1)) // NW

        def chunk_body(i, carry):
            base = pl.multiple_of(i * CHUNK, CHUNK)
            pltpu.sync_copy(src_h.at[pl.ds(base, CHUNK)], src_v)
            pltpu.sync_copy(dst_h.at[pl.ds(base, CHUNK)], dst_v)
            pltpu.async_copy(x_h.at[src_v], rows_v, sem).wait()
            pltpu.sync_copy(rows_v, summed_sh.at[dst_v], add=True)
            pltpu.sync_copy(ones_v, cnt_sh.at[dst_v], add=True)
            return carry

        lax.fori_loop(c0, c1, chunk_body, 0)
        plsc.subcore_barrier()

        @pl.when(s < NS - 1)
        def _pub_full():
            pltpu.sync_copy(summed_sh.at[pl.ds(r0, stripe)],
                            summed_o.at[c, pl.ds(r0, stripe)])
            pltpu.sync_copy(cnt_sh.at[pl.ds(r0, stripe)],
                            cnt_o.at[c, pl.ds(r0, stripe)])

        @pl.when(s == NS - 1)
        def _pub_tail():
            t0 = (NS - 1) * stripe
            pltpu.sync_copy(summed_sh.at[pl.ds(t0, s_tail)],
                            summed_o.at[c, pl.ds(t0, s_tail)])
            pltpu.sync_copy(cnt_sh.at[pl.ds(t0, s_tail)],
                            cnt_o.at[c, pl.ds(t0, s_tail)])

        @pl.when(wid < NW - 1)
        def _full():
            b = pl.multiple_of(wid * 128, 128)
            pltpu.sync_copy(batch_h.at[pl.ds(b, 128)], src_v)
            pltpu.async_copy(attr_h.at[src_v], rows_v, sem).wait()
            pltpu.sync_copy(rows_v, attr_o.at[pl.ds(b, 128)])

        @pl.when(wid == NW - 1)
        def _tail():
            b = (NW - 1) * 128
            pltpu.sync_copy(batch_h.at[pl.ds(b, a_tail)],
                            src_v.at[pl.ds(0, a_tail)])
            pltpu.async_copy(attr_h.at[src_v.at[pl.ds(0, a_tail)]],
                             rows_v.at[pl.ds(0, a_tail)], sem).wait()
            pltpu.sync_copy(rows_v.at[pl.ds(0, a_tail)],
                            attr_o.at[pl.ds(b, a_tail)])

    return body(x, src, dst, batch, attr, zs, zc, ones)


def _tc_combine(summed2, cnt2, x_t, attr_g, W_l, W_r, W_lin, b_l, b_lin):
    n_dst, d = x_t.shape
    blk = 1000
    grid = n_dst // blk
    dn = (((1,), (1,)), ((), ()))

    def body(s2, c2, xt, ag, wl, wr, wlin, bl, blin, o):
        ssum = s2[0] + s2[1]
        cnt = c2[0] + c2[1]
        mean = ssum / jnp.maximum(cnt[:, 0:1], 1.0)
        acc = lax.dot_general(mean, wl[...], dn,
                              preferred_element_type=jnp.float32)
        acc = acc + lax.dot_general(xt[...], wr[...], dn,
                                    preferred_element_type=jnp.float32)
        acc = acc + 0.25 * lax.dot_general(ag[...], wlin[...], dn,
                                           preferred_element_type=jnp.float32)
        acc = acc + (bl[...] + 0.25 * blin[...])
        o[...] = jnp.maximum(acc, 0.0)

    return pl.pallas_call(
        body,
        grid=(grid,),
        in_specs=[
            pl.BlockSpec((NC, blk, d), lambda i: (0, i, 0)),
            pl.BlockSpec((NC, blk, d), lambda i: (0, i, 0)),
            pl.BlockSpec((blk, d), lambda i: (i, 0)),
            pl.BlockSpec((blk, d), lambda i: (i, 0)),
            pl.BlockSpec((d, d), lambda i: (0, 0)),
            pl.BlockSpec((d, d), lambda i: (0, 0)),
            pl.BlockSpec((d, d), lambda i: (0, 0)),
            pl.BlockSpec((1, d), lambda i: (0, 0)),
            pl.BlockSpec((1, d), lambda i: (0, 0)),
        ],
        out_specs=pl.BlockSpec((blk, d), lambda i: (i, 0)),
        out_shape=jax.ShapeDtypeStruct((n_dst, d), jnp.float32),
    )(summed2, cnt2, x_t, attr_g, W_l, W_r, W_lin, b_l, b_lin)


def kernel(x, edge_index, batch, attr, W_l, b_l, W_r, W_lin, b_lin,
           size_src, size_dst):
    src = edge_index[0]
    dst = edge_index[1]
    n_dst = batch.shape[0]
    zs = jnp.zeros((n_dst, x.shape[1]), jnp.float32)
    zc = jnp.zeros((n_dst, x.shape[1]), jnp.float32)
    ones = jnp.ones((CHUNK, x.shape[1]), jnp.float32)
    summed2, cnt2, attr_g = _sc_agg(x, src, dst, batch, attr, zs, zc, ones)
    return _tc_combine(summed2, cnt2, x[:n_dst], attr_g, W_l, W_r, W_lin,
                       b_l.reshape(1, -1), b_lin.reshape(1, -1))
